# Initial kernel scaffold; baseline (speedup 1.0000x reference)
import jax
import jax.numpy as jnp
from jax.experimental import pallas as pl


def kernel(pred_logits, pred_boxes, tgt_labels, tgt_boxes, longscore):
    raise NotImplementedError("write your pallas kernel here")


# speedup vs baseline: 40.1062x; 40.1062x over previous
import functools

import jax
import jax.numpy as jnp
from jax import lax
from jax.experimental import pallas as pl
from jax.experimental.pallas import tpu as pltpu

_N_EXTRACT_IOU = 10
_N_EXTRACT_COST = 11


def _row_chunk(n):
    for c in (2000, 1000, 500):
        if n % c == 0 and n > c:
            return c
    return n


def _cost_iou_kernel(lg_ref, bx_ref, lab_ref, gt_ref, cost_ref, iou_ref):
    lg = lg_ref[0]
    labels = lab_ref[0]
    C = lg.shape[1]
    G = labels.shape[1]
    onehot = (lax.broadcasted_iota(jnp.int32, (C, G), 0) == labels).astype(jnp.float32)
    gl = jnp.dot(lg, onehot, preferred_element_type=jnp.float32,
                 precision=lax.Precision.HIGHEST)
    prob = jax.nn.sigmoid(gl)
    neg = (0.75 * (prob * prob)) * (-jnp.log(1.0 - prob))
    pos = (0.25 * ((1.0 - prob) * (1.0 - prob))) * (-jnp.log(prob + 1e-8))
    cost_class = pos - neg

    bx = bx_ref[0]
    cx, cy = bx[:, 0:1], bx[:, 1:2]
    w, h = bx[:, 2:3], bx[:, 3:4]
    qx1, qy1 = cx - 0.5 * w, cy - 0.5 * h
    qx2, qy2 = cx + 0.5 * w, cy + 0.5 * h

    gt = gt_ref[0]
    gcx, gcy = gt[0:1, :], gt[1:2, :]
    gw, gh = gt[2:3, :], gt[3:4, :]
    gx1, gy1 = gcx - 0.5 * gw, gcy - 0.5 * gh
    gx2, gy2 = gcx + 0.5 * gw, gcy + 0.5 * gh

    area1 = (qx2 - qx1) * (qy2 - qy1)
    area2 = (gx2 - gx1) * (gy2 - gy1)
    ltx = jnp.maximum(qx1, gx1)
    lty = jnp.maximum(qy1, gy1)
    rbx = jnp.minimum(qx2, gx2)
    rby = jnp.minimum(qy2, gy2)
    iw = jnp.maximum(rbx - ltx, 0.0)
    ih = jnp.maximum(rby - lty, 0.0)
    inter = iw * ih
    union = area1 + area2 - inter
    iou = inter / (union + 1e-8)

    eltx = jnp.minimum(qx1, gx1)
    elty = jnp.minimum(qy1, gy1)
    erbx = jnp.maximum(qx2, gx2)
    erby = jnp.maximum(qy2, gy2)
    ew = jnp.maximum(erbx - eltx, 0.0)
    eh = jnp.maximum(erby - elty, 0.0)
    earea = ew * eh
    giou = iou - (earea - union) / (earea + 1e-8)
    cost_giou = -giou

    in_boxes = (cx > gx1) & (cx < gx2) & (cy > gy1) & (cy < gy2)
    in_boxes_all = jnp.sum(in_boxes.astype(jnp.int32), axis=1, keepdims=True) > 0
    cr = 2.5 / 32.0
    in_centers = ((cx > (gcx - cr)) & (cx < (gcx + cr))
                  & (cy > (gcy - cr)) & (cy < (gcy + cr)))
    in_centers_all = jnp.sum(in_centers.astype(jnp.int32), axis=1, keepdims=True) > 0
    fg = in_boxes_all | in_centers_all
    in_bc = in_boxes & in_centers

    cost = cost_class + 3.0 * cost_giou
    cost = cost + 100.0 * (~in_bc).astype(jnp.float32)
    cost = cost + 10000.0 * (~fg).astype(jnp.float32)

    cost_ref[0] = cost
    iou_ref[0] = iou


def _topk_kernel(cost_ref, iou_ref, bv_ref, br_ref, w_ref):
    N, G = w_ref.shape
    RCH = _row_chunk(N)
    NCH = N // RCH
    chunk_iota = lax.broadcasted_iota(jnp.int32, (RCH, G), 0)

    def copy_in(src_ref):
        def body(c, _):
            w_ref[pl.ds(c * RCH, RCH), :] = src_ref[0, pl.ds(c * RCH, RCH), :]
            return 0
        lax.fori_loop(0, NCH, body, 0)

    def extract(sign, fill):
        def body(c, carry):
            m, r = carry
            x = w_ref[pl.ds(c * RCH, RCH), :]
            ids = chunk_iota + c * RCH
            if sign > 0:
                cm = jnp.max(x, axis=0, keepdims=True)
                better = cm > m
            else:
                cm = jnp.min(x, axis=0, keepdims=True)
                better = cm < m
            cmf = jnp.broadcast_to(cm, x.shape)
            ci = jnp.min(jnp.where(x == cmf, ids, N), axis=0, keepdims=True)
            take = better | ((cm == m) & (ci < r))
            return jnp.where(take, cm, m), jnp.where(take, ci, r)
        init = (jnp.full((1, G), fill, jnp.float32), jnp.full((1, G), N, jnp.int32))
        return lax.fori_loop(0, NCH, body, init)

    def mask_out(r, fill):
        def body(c, _):
            ids = chunk_iota + c * RCH
            x = w_ref[pl.ds(c * RCH, RCH), :]
            rf = jnp.broadcast_to(r, x.shape)
            w_ref[pl.ds(c * RCH, RCH), :] = jnp.where(ids == rf, fill, x)
            return 0
        lax.fori_loop(0, NCH, body, 0)

    copy_in(iou_ref)
    s = jnp.zeros((1, G), jnp.float32)
    for j in range(_N_EXTRACT_IOU):
        m, r = extract(+1, -jnp.inf)
        s = s + m
        if j + 1 < _N_EXTRACT_IOU:
            mask_out(r, -jnp.inf)
    ks = jnp.maximum(s.astype(jnp.int32), 1)

    copy_in(cost_ref)
    vals, rows = [], []
    for j in range(_N_EXTRACT_COST):
        m, r = extract(-1, jnp.inf)
        vals.append(m)
        rows.append(r)
        if j + 1 < _N_EXTRACT_COST:
            mask_out(r, jnp.inf)

    bv = jnp.zeros((1, G), jnp.float32)
    br = jnp.zeros((1, G), jnp.int32)
    for j in range(1, _N_EXTRACT_COST):
        sel = ks == j
        bv = jnp.where(sel, vals[j], bv)
        br = jnp.where(sel, rows[j], br)
    bv_ref[0] = bv
    br_ref[0] = br


def _match_kernel(nblocks, cost_ref, iou_ref, bv_ref, br_ref,
                  match_ref, iougt_ref, stats_ref):
    n = pl.program_id(1)
    c = cost_ref[0]
    io = iou_ref[0]
    RB, G = c.shape
    bv = bv_ref[0]
    br = br_ref[0]

    ids = lax.broadcasted_iota(jnp.int32, (RB, G), 0) + n * RB
    bvf = jnp.broadcast_to(bv, c.shape)
    brf = jnp.broadcast_to(br, c.shape)
    m1 = (c < bvf) | ((c == bvf) & (ids < brf))
    anchor = jnp.sum(m1.astype(jnp.float32), axis=1, keepdims=True)

    rm = jnp.min(c, axis=1, keepdims=True)
    colids = lax.broadcasted_iota(jnp.int32, (RB, G), 1)
    rmf = jnp.broadcast_to(rm, c.shape)
    am = jnp.min(jnp.where(c == rmf, colids, G), axis=1, keepdims=True)
    keep = (colids == jnp.broadcast_to(am, c.shape)).astype(jnp.float32)
    sel = jnp.broadcast_to(anchor > 1.0, c.shape)
    mf = jnp.where(sel, keep, m1.astype(jnp.float32))
    match_ref[0] = mf

    @pl.when(n == 0)
    def _():
        stats_ref[...] = jnp.zeros_like(stats_ref)

    stats_ref[0:1, :] = stats_ref[0:1, :] + jnp.sum(mf, axis=0, keepdims=True)
    stats_ref[1:2, :] = stats_ref[1:2, :] + jnp.sum(mf * io, axis=0, keepdims=True)

    @pl.when(n == nblocks - 1)
    def _():
        cnt = stats_ref[0:1, :]
        sm = stats_ref[1:2, :]
        iougt_ref[0] = jnp.where(
            cnt > 0, jnp.sqrt(sm / jnp.maximum(cnt, 1.0)), 0.001)


def kernel(pred_logits, pred_boxes, tgt_labels, tgt_boxes, longscore):
    B, N, C = pred_logits.shape
    G = tgt_labels.shape[1]
    RB = _row_chunk(N)
    NB = N // RB

    labels3 = tgt_labels.reshape(B, 1, G).astype(jnp.int32)
    gt_t = jnp.transpose(tgt_boxes, (0, 2, 1))

    f32 = jnp.float32
    cost, ious = pl.pallas_call(
        _cost_iou_kernel,
        grid=(B, NB),
        in_specs=[
            pl.BlockSpec((1, RB, C), lambda b, n: (b, n, 0)),
            pl.BlockSpec((1, RB, 4), lambda b, n: (b, n, 0)),
            pl.BlockSpec((1, 1, G), lambda b, n: (b, 0, 0)),
            pl.BlockSpec((1, 4, G), lambda b, n: (b, 0, 0)),
        ],
        out_specs=[
            pl.BlockSpec((1, RB, G), lambda b, n: (b, n, 0)),
            pl.BlockSpec((1, RB, G), lambda b, n: (b, n, 0)),
        ],
        out_shape=[
            jax.ShapeDtypeStruct((B, N, G), f32),
            jax.ShapeDtypeStruct((B, N, G), f32),
        ],
    )(pred_logits, pred_boxes, labels3, gt_t)

    bound_v, bound_r = pl.pallas_call(
        _topk_kernel,
        grid=(B,),
        in_specs=[
            pl.BlockSpec((1, N, G), lambda b: (b, 0, 0)),
            pl.BlockSpec((1, N, G), lambda b: (b, 0, 0)),
        ],
        out_specs=[
            pl.BlockSpec((1, 1, G), lambda b: (b, 0, 0)),
            pl.BlockSpec((1, 1, G), lambda b: (b, 0, 0)),
        ],
        out_shape=[
            jax.ShapeDtypeStruct((B, 1, G), f32),
            jax.ShapeDtypeStruct((B, 1, G), jnp.int32),
        ],
        scratch_shapes=[pltpu.VMEM((N, G), f32)],
    )(cost, ious)

    matching, iou_gt = pl.pallas_call(
        functools.partial(_match_kernel, NB),
        grid=(B, NB),
        in_specs=[
            pl.BlockSpec((1, RB, G), lambda b, n: (b, n, 0)),
            pl.BlockSpec((1, RB, G), lambda b, n: (b, n, 0)),
            pl.BlockSpec((1, 1, G), lambda b, n: (b, 0, 0)),
            pl.BlockSpec((1, 1, G), lambda b, n: (b, 0, 0)),
        ],
        out_specs=[
            pl.BlockSpec((1, RB, G), lambda b, n: (b, n, 0)),
            pl.BlockSpec((1, 1, G), lambda b, n: (b, 0, 0)),
        ],
        out_shape=[
            jax.ShapeDtypeStruct((B, N, G), f32),
            jax.ShapeDtypeStruct((B, 1, G), f32),
        ],
        scratch_shapes=[pltpu.VMEM((8, G), f32)],
    )(cost, ious, bound_v, bound_r)

    return cost, matching, iou_gt.reshape(B, G)
